# Initial kernel scaffold; baseline (speedup 1.0000x reference)
import jax
import jax.numpy as jnp
from jax.experimental import pallas as pl


def kernel(edge_index, emb, Wxz, bxz, Whz, bhz, Wxr, bxr, Whr, bhr, Wxh, bxh, Whh, bhh, W0, b0, W1, b1, W2, b2, W3, b3, W4, b4, W5, b5):
    raise NotImplementedError("write your pallas kernel here")


# speedup vs baseline: 5.6332x; 5.6332x over previous
import functools

import jax
import jax.numpy as jnp
from jax import lax
from jax.experimental import pallas as pl
from jax.experimental.pallas import tpu as pltpu
from jax.experimental.pallas import tpu_sc as plsc

N = 100000
E = 1600000
EMB = 128
OUT = 16
TGT = 3

NC = 2
NS = 16
NW = NC * NS

EW = E // NW
GE = 400
NG = EW // GE
CH = 80
NCH = GE // CH

NBLK = 2000
RBLK = 2000


def _node_body(x_ref, wz_ref, bz_ref, wh_ref, bh_ref, wf_ref, wf2_ref,
               t_ref, t2_ref):
    x = x_ref[...]
    z = jax.nn.sigmoid(
        jnp.dot(x, wz_ref[...], preferred_element_type=jnp.float32) + bz_ref[...])
    ht = jnp.tanh(
        jnp.dot(x, wh_ref[...], preferred_element_type=jnp.float32) + bh_ref[...])
    h = jnp.maximum((1.0 - z) * ht, 0.0)
    t_ref[...] = jnp.dot(h, wf_ref[...], preferred_element_type=jnp.float32)
    t2_ref[...] = jnp.dot(h, wf2_ref[...], preferred_element_type=jnp.float32)


def _node_stage(emb, wz, bz, wh, bh, wf, wf2):
    grid = (N // NBLK,)
    return pl.pallas_call(
        _node_body,
        grid=grid,
        in_specs=[
            pl.BlockSpec((NBLK, EMB), lambda i: (i, 0)),
            pl.BlockSpec((EMB, OUT), lambda i: (0, 0)),
            pl.BlockSpec((1, OUT), lambda i: (0, 0)),
            pl.BlockSpec((EMB, OUT), lambda i: (0, 0)),
            pl.BlockSpec((1, OUT), lambda i: (0, 0)),
            pl.BlockSpec((OUT, OUT), lambda i: (0, 0)),
            pl.BlockSpec((OUT, OUT), lambda i: (0, 0)),
        ],
        out_specs=[
            pl.BlockSpec((NBLK, OUT), lambda i: (i, 0)),
            pl.BlockSpec((NBLK, OUT), lambda i: (i, 0)),
        ],
        out_shape=[
            jax.ShapeDtypeStruct((N, OUT), jnp.float32),
            jax.ShapeDtypeStruct((N, OUT), jnp.float32),
        ],
    )(emb, wz, bz, wh, bh, wf, wf2)


def _sc_body(t_hbm, t2_hbm, home_hbm, away_hbm, b0_hbm, out_hbm,
             idx_h, idx_a, rows_h, rows_a, out_v, b0_v, sem):
    wid = lax.axis_index("s") * NC + lax.axis_index("c")
    base = wid * EW
    pltpu.sync_copy(b0_hbm, b0_v)
    lanes = lax.iota(jnp.int32, 16)
    shift = (lanes + 8) % 16
    lo_mask = lanes < 8

    def group(g, carry):
        off = base + g * GE
        pltpu.sync_copy(home_hbm.at[pl.ds(off, GE)], idx_h)
        pltpu.sync_copy(away_hbm.at[pl.ds(off, GE)], idx_a)
        copies = []
        for c in range(NCH):
            copies.append(pltpu.async_copy(
                t_hbm.at[idx_h.at[pl.ds(c * CH, CH)]],
                rows_h.at[pl.ds(c * CH, CH)], sem))
            copies.append(pltpu.async_copy(
                t2_hbm.at[idx_a.at[pl.ds(c * CH, CH)]],
                rows_a.at[pl.ds(c * CH, CH)], sem))
        for cp in copies:
            cp.wait()

        def inner(j, carry2):
            e = j * 2
            o_a = rows_h[e, :] + rows_a[e, :]
            o_b = rows_h[e + 1, :] + rows_a[e + 1, :]
            o_bs = jnp.take_along_axis(o_b, shift, axis=0)
            merged = jnp.where(lo_mask, o_a, o_bs)
            out_v[pl.ds(j * 16, 16)] = jnp.maximum(
                merged + b0_v[...], jnp.zeros((16,), jnp.float32))
            return carry2

        lax.fori_loop(0, GE * 8 // 16, inner, 0, unroll=4)
        pltpu.sync_copy(out_v, out_hbm.at[pl.ds(off * 8, GE * 8)])
        return carry

    lax.fori_loop(0, NG, group, 0)


def _sc_gather(t, t2, home, away, b0t):
    mesh = plsc.VectorSubcoreMesh(core_axis_name="c", subcore_axis_name="s")
    fn = pl.kernel(
        _sc_body,
        out_type=jax.ShapeDtypeStruct((E * 8,), jnp.float32),
        mesh=mesh,
        scratch_types=[
            pltpu.VMEM((GE,), jnp.int32),
            pltpu.VMEM((GE,), jnp.int32),
            pltpu.VMEM((GE, OUT), jnp.float32),
            pltpu.VMEM((GE, OUT), jnp.float32),
            pltpu.VMEM((GE * 8,), jnp.float32),
            pltpu.VMEM((16,), jnp.float32),
            pltpu.SemaphoreType.DMA,
        ],
        compiler_params=pltpu.CompilerParams(use_tc_tiling_on_sc=False),
    )
    return fn(t, t2, home, away, b0t)


def _mlp_body(x_ref, w1, b1, w2, b2, w3, b3, w4, b4, w5, b5, bs, o_ref):
    x = x_ref[...]
    for w, b in ((w1, b1), (w2, b2), (w3, b3), (w4, b4)):
        x = jnp.maximum(
            jnp.dot(x, w[...], preferred_element_type=jnp.float32) + b[...], 0.0)
    x5 = jnp.maximum(
        jnp.dot(x, w5[...], preferred_element_type=jnp.float32) + b5[...], 0.0)
    ex = jnp.exp(x5)
    denom = jnp.dot(ex, bs[...], preferred_element_type=jnp.float32)
    o_ref[...] = ex / denom


def _mlp_stage(s2, wd, bd, wd5, bd5, bsum):
    grid = ((E // 16) // RBLK,)
    full = lambda shape: pl.BlockSpec(shape, lambda i: (0, 0))
    return pl.pallas_call(
        _mlp_body,
        grid=grid,
        in_specs=[
            pl.BlockSpec((RBLK, 128), lambda i: (i, 0)),
            full((128, 128)), full((1, 128)),
            full((128, 128)), full((1, 128)),
            full((128, 128)), full((1, 128)),
            full((128, 128)), full((1, 128)),
            full((128, 48)), full((1, 48)),
            full((48, 48)),
        ],
        out_specs=pl.BlockSpec((RBLK, 48), lambda i: (i, 0)),
        out_shape=jax.ShapeDtypeStruct((E // 16, 48), jnp.float32),
    )(s2, wd[0], bd[0], wd[1], bd[1], wd[2], bd[2], wd[3], bd[3],
      wd5, bd5, bsum)


def kernel(edge_index, emb, Wxz, bxz, Whz, bhz, Wxr, bxr, Whr, bhr,
           Wxh, bxh, Whh, bhh, W0, b0, W1, b1, W2, b2, W3, b3, W4, b4,
           W5, b5):
    home = edge_index[0]
    away = edge_index[1]

    bz = (bxz + bhz).reshape(1, OUT)
    bh = (bxh + bhh).reshape(1, OUT)
    wfold = jnp.concatenate([W0[:OUT, :], W0[OUT:, :]], axis=1)
    wfold2 = jnp.concatenate([W0[OUT:, :], W0[:OUT, :]], axis=1)
    t, t2 = _node_stage(emb, Wxz, bz, Wxh, bh, wfold, wfold2)

    b0t = jnp.concatenate([b0, b0])
    s_flat = _sc_gather(t, t2, home, away, b0t)
    s2 = s_flat.reshape(E // 16, 128)

    eye = jnp.eye(16, dtype=jnp.float32)
    wd = [jnp.kron(eye, W) for W in (W1, W2, W3, W4)]
    bd = [jnp.tile(b, 16).reshape(1, 128) for b in (b1, b2, b3, b4)]
    wd5 = jnp.kron(eye, W5)
    bd5 = jnp.tile(b5, 16).reshape(1, 48)
    bsum = jnp.kron(eye, jnp.ones((TGT, TGT), jnp.float32))

    o48 = _mlp_stage(s2, wd, bd, wd5, bd5, bsum)
    return o48.reshape(E, TGT)
